# Initial kernel scaffold; baseline (speedup 1.0000x reference)
import functools

import jax
import jax.numpy as jnp
from jax import lax
from jax.experimental import pallas as pl
from jax.experimental.pallas import tpu as pltpu
from jax.experimental.pallas import tpu_sc as plsc

N = 10000
E = 320000
D = 128

NC = 2
NS = 16
NW = NC * NS

C = 128
NCH = 80
EPT = C * NCH
EPAD = EPT * NW

ROWS_PER_TILE = N // NS


def _sc_body(x_hbm, col_hbm, row_hbm, w_hbm, zeros_hbm, out_hbm,
             col_v, row_v, w_v, rows_v, agg_sh, sem):
    c = lax.axis_index("c")
    s = lax.axis_index("s")
    wid = c * NS + s

    pltpu.sync_copy(zeros_hbm.at[pl.ds(s * ROWS_PER_TILE, ROWS_PER_TILE)],
                    agg_sh.at[pl.ds(s * ROWS_PER_TILE, ROWS_PER_TILE)])

    pltpu.sync_copy(col_hbm.at[wid], col_v)
    pltpu.sync_copy(row_hbm.at[wid], row_v)
    pltpu.sync_copy(w_hbm.at[wid], w_v)

    plsc.subcore_barrier()

    @pl.loop(0, NCH)
    def _chunk(j):
        pltpu.async_copy(x_hbm.at[col_v.at[j]], rows_v, sem).wait()
        jv = jnp.full((16,), j, jnp.int32)
        for e in range(C):
            wb = plsc.load_gather(w_v, [jv, jnp.full((16,), e, jnp.int32)])
            for f in range(D // 16):
                sl = pl.ds(f * 16, 16)
                rows_v[e, sl] = rows_v[e, sl] * wb
        pltpu.sync_copy(rows_v, agg_sh.at[row_v.at[j]], add=True)

    plsc.subcore_barrier()

    pltpu.sync_copy(agg_sh.at[pl.ds(s * ROWS_PER_TILE, ROWS_PER_TILE)],
                    out_hbm.at[c, pl.ds(s * ROWS_PER_TILE, ROWS_PER_TILE)])


@jax.jit
def _sc_aggregate(x, col3, row3, w3, zeros):
    mesh = plsc.VectorSubcoreMesh(core_axis_name="c", subcore_axis_name="s")
    return pl.kernel(
        _sc_body,
        out_type=jax.ShapeDtypeStruct((NC, N, D), jnp.float32),
        mesh=mesh,
        scratch_types=[
            pltpu.VMEM((NCH, C), jnp.int32),
            pltpu.VMEM((NCH, C), jnp.int32),
            pltpu.VMEM((NCH, C), jnp.float32),
            pltpu.VMEM((C, D), jnp.float32),
            pltpu.VMEM_SHARED((N, D), jnp.float32),
            pltpu.SemaphoreType.DMA,
        ],
    )(x, col3, row3, w3, zeros)


def _tc_body(p_ref, w_ref, o_ref):
    acc = p_ref[0] + p_ref[1]
    o_ref[...] = jnp.maximum(
        jnp.dot(acc, w_ref[...], preferred_element_type=jnp.float32), 0.0)


@jax.jit
def _tc_combine(p, W):
    bm = 1000
    return pl.pallas_call(
        _tc_body,
        grid=(N // bm,),
        in_specs=[
            pl.BlockSpec((NC, bm, D), lambda i: (0, i, 0)),
            pl.BlockSpec((D, D), lambda i: (0, 0)),
        ],
        out_specs=pl.BlockSpec((bm, D), lambda i: (i, 0)),
        out_shape=jax.ShapeDtypeStruct((N, D), jnp.float32),
    )(p, W)


def kernel(x, edge_index, edge_weight, W):
    row = edge_index[0]
    col = edge_index[1]
    pad = EPAD - E
    col_p = jnp.concatenate([col, jnp.zeros((pad,), jnp.int32)])
    row_p = jnp.concatenate([row, jnp.zeros((pad,), jnp.int32)])
    w_p = jnp.concatenate([edge_weight, jnp.zeros((pad,), jnp.float32)])
    col3 = col_p.reshape(NW, NCH, C)
    row3 = row_p.reshape(NW, NCH, C)
    w3 = w_p.reshape(NW, NCH, C)
    zeros = jnp.zeros((N, D), jnp.float32)
    partials = _sc_aggregate(x, col3, row3, w3, zeros)
    return _tc_combine(partials, W)


# speedup vs baseline: 2.8895x; 2.8895x over previous
import functools

import jax
import jax.numpy as jnp
from jax import lax
from jax.experimental import pallas as pl
from jax.experimental.pallas import tpu as pltpu
from jax.experimental.pallas import tpu_sc as plsc

N = 10000
E = 320000
D = 128

NC = 2
NS = 16
NW = NC * NS

C = 128
NCH = 80
EPT = C * NCH
EPAD = EPT * NW

NPAD = 10240
ROWS_PER_TILE = NPAD // NS

_GATHER_DNUMS = lax.GatherDimensionNumbers(
    offset_dims=(), collapsed_slice_dims=(0,), start_index_map=(0,))


def _sc_body(x_hbm, col_hbm, row_hbm, w_hbm, zeros_hbm, out_hbm,
             col_v, row_v, w_v, rows_v, agg_sh, sem):
    c = lax.axis_index("c")
    s = lax.axis_index("s")
    wid = c * NS + s

    pltpu.sync_copy(zeros_hbm.at[pl.ds(s * ROWS_PER_TILE, ROWS_PER_TILE)],
                    agg_sh.at[pl.ds(s * ROWS_PER_TILE, ROWS_PER_TILE)])

    pltpu.sync_copy(col_hbm.at[wid], col_v)
    pltpu.sync_copy(row_hbm.at[wid], row_v)
    pltpu.sync_copy(w_hbm.at[wid], w_v)

    plsc.subcore_barrier()

    @pl.loop(0, NCH)
    def _chunk(j):
        pltpu.async_copy(x_hbm.at[col_v.at[j]], rows_v, sem).wait()
        jbase = j * C
        for g in range(C // 16):
            wvec = w_v[pl.ds(jbase + g * 16, 16)]
            for l in range(16):
                e = g * 16 + l
                wb = lax.gather(
                    wvec, jnp.full((16, 1), l, jnp.int32),
                    _GATHER_DNUMS, slice_sizes=(1,),
                    mode=lax.GatherScatterMode.PROMISE_IN_BOUNDS)
                for f in range(D // 16):
                    sl = pl.ds(f * 16, 16)
                    rows_v[e, sl] = rows_v[e, sl] * wb
        pltpu.sync_copy(rows_v, agg_sh.at[row_v.at[j]], add=True)

    plsc.subcore_barrier()

    pltpu.sync_copy(agg_sh.at[pl.ds(s * ROWS_PER_TILE, ROWS_PER_TILE)],
                    out_hbm.at[c, pl.ds(s * ROWS_PER_TILE, ROWS_PER_TILE)])


@jax.jit
def _sc_aggregate(x, col3, row3, w3, zeros):
    mesh = plsc.VectorSubcoreMesh(core_axis_name="c", subcore_axis_name="s")
    return pl.kernel(
        _sc_body,
        out_type=jax.ShapeDtypeStruct((NC, NPAD, D), jnp.float32),
        mesh=mesh,
        scratch_types=[
            pltpu.VMEM((NCH, C), jnp.int32),
            pltpu.VMEM((NCH, C), jnp.int32),
            pltpu.VMEM((EPT,), jnp.float32),
            pltpu.VMEM((C, D), jnp.float32),
            pltpu.VMEM_SHARED((NPAD, D), jnp.float32),
            pltpu.SemaphoreType.DMA,
        ],
    )(x, col3, row3, w3, zeros)


def _tc_body(p_ref, w_ref, o_ref):
    acc = p_ref[0] + p_ref[1]
    o_ref[...] = jnp.maximum(
        jnp.dot(acc, w_ref[...], preferred_element_type=jnp.float32), 0.0)


@jax.jit
def _tc_combine(p, W):
    bm = 1000
    return pl.pallas_call(
        _tc_body,
        grid=(N // bm,),
        in_specs=[
            pl.BlockSpec((NC, bm, D), lambda i: (0, i, 0)),
            pl.BlockSpec((D, D), lambda i: (0, 0)),
        ],
        out_specs=pl.BlockSpec((bm, D), lambda i: (i, 0)),
        out_shape=jax.ShapeDtypeStruct((N, D), jnp.float32),
    )(p, W)


def kernel(x, edge_index, edge_weight, W):
    row = edge_index[0]
    col = edge_index[1]
    pad = EPAD - E
    col_p = jnp.concatenate([col, jnp.zeros((pad,), jnp.int32)])
    row_p = jnp.concatenate([row, jnp.zeros((pad,), jnp.int32)])
    w_p = jnp.concatenate([edge_weight, jnp.zeros((pad,), jnp.float32)])
    col3 = col_p.reshape(NW, NCH, C)
    row3 = row_p.reshape(NW, NCH, C)
    w3 = w_p.reshape(NW, EPT)
    zeros = jnp.zeros((NPAD, D), jnp.float32)
    partials = _sc_aggregate(x, col3, row3, w3, zeros)
    return _tc_combine(partials[:, :N, :], W)


# speedup vs baseline: 3.3275x; 1.1516x over previous
import jax
import jax.numpy as jnp
from jax import lax
from jax.experimental import pallas as pl
from jax.experimental.pallas import tpu as pltpu
from jax.experimental.pallas import tpu_sc as plsc

N = 10000
E = 320000
D = 128
NC = 2
NS = 16
NW = NC * NS

C = 128
NCH = 80
EPT = C * NCH
EPAD = EPT * NW

NPAD = 10240
ROWS_PER_TILE = NPAD // NS

_GATHER_DNUMS = lax.GatherDimensionNumbers(
    offset_dims=(), collapsed_slice_dims=(0,), start_index_map=(0,))


def _scale_chunk(rows_b, w2, b):
    @pl.loop(0, C // 16)
    def _g(g):
        base = g * 16
        wvec = w2[b, pl.ds(base, 16)]
        for l in range(16):
            wb = lax.gather(
                wvec, jnp.full((16, 1), l, jnp.int32),
                _GATHER_DNUMS, slice_sizes=(1,),
                mode=lax.GatherScatterMode.PROMISE_IN_BOUNDS)
            for f in range(D // 16):
                sl = pl.ds(f * 16, 16)
                rows_b[base + l, sl] = rows_b[base + l, sl] * wb


def _edge_load(col_hbm, row_hbm, w_hbm, col2, row2, w2, wid, k, b, es):
    pltpu.async_copy(col_hbm.at[wid].at[k], col2.at[b], es)
    pltpu.async_copy(row_hbm.at[wid].at[k], row2.at[b], es)
    pltpu.async_copy(w_hbm.at[wid].at[k], w2.at[b], es)


def _edge_wait(col_hbm, col2, row2, w2, wid, b, es):
    pltpu.make_async_copy(col_hbm.at[0].at[0], col2.at[b], es).wait()
    pltpu.make_async_copy(col_hbm.at[0].at[0], row2.at[b], es).wait()
    pltpu.make_async_copy(col_hbm.at[0].at[0], w2.at[b], es).wait()


def _sc_body(x_hbm, col_hbm, row_hbm, w_hbm, zeros_hbm, out_hbm,
             col2, row2, w2, rows0, rows1, agg_sh, gs0, gs1, es0, es1):
    c = lax.axis_index("c")
    s = lax.axis_index("s")
    wid = c * NS + s

    pltpu.sync_copy(zeros_hbm.at[pl.ds(s * ROWS_PER_TILE, ROWS_PER_TILE)],
                    agg_sh.at[pl.ds(s * ROWS_PER_TILE, ROWS_PER_TILE)])

    plsc.subcore_barrier()

    xh = x_hbm
    ess = (es0, es1)

    _edge_load(col_hbm, row_hbm, w_hbm, col2, row2, w2, wid, 0, 0, es0)
    _edge_load(col_hbm, row_hbm, w_hbm, col2, row2, w2, wid, 1, 1, es1)
    _edge_wait(col_hbm, col2, row2, w2, wid, 0, es0)
    pltpu.async_copy(xh.at[col2.at[0]], rows0, gs0)

    @pl.loop(0, NCH, step=2)
    def _outer(j):
        for b in range(2):
            k = j + b
            o = 1 - b
            rows_b, rows_o = (rows0, rows1) if b == 0 else (rows1, rows0)
            gs_b, gs_o = (gs0, gs1) if b == 0 else (gs1, gs0)

            pltpu.make_async_copy(xh.at[col2.at[b]], rows_b, gs_b).wait()

            @pl.when(k + 1 < NCH)
            def _():
                _edge_wait(col_hbm, col2, row2, w2, wid, o, ess[o])
                pltpu.async_copy(xh.at[col2.at[o]], rows_o, gs_o)

            _scale_chunk(rows_b, w2, b)

            pltpu.sync_copy(rows_b, agg_sh.at[row2.at[b]], add=True)

            @pl.when(k + 2 < NCH)
            def _():
                _edge_load(col_hbm, row_hbm, w_hbm, col2, row2, w2,
                           wid, k + 2, b, ess[b])

    plsc.subcore_barrier()

    pltpu.sync_copy(agg_sh.at[pl.ds(s * ROWS_PER_TILE, ROWS_PER_TILE)],
                    out_hbm.at[c, pl.ds(s * ROWS_PER_TILE, ROWS_PER_TILE)])


@jax.jit
def _sc_aggregate(x, col3, row3, w3, zeros):
    mesh = plsc.VectorSubcoreMesh(core_axis_name="c", subcore_axis_name="s")
    return pl.kernel(
        _sc_body,
        out_type=jax.ShapeDtypeStruct((NC, NPAD, D), jnp.float32),
        mesh=mesh,
        scratch_types=[
            pltpu.VMEM((2, C), jnp.int32),
            pltpu.VMEM((2, C), jnp.int32),
            pltpu.VMEM((2, C), jnp.float32),
            pltpu.VMEM((C, D), jnp.float32),
            pltpu.VMEM((C, D), jnp.float32),
            pltpu.VMEM_SHARED((NPAD, D), jnp.float32),
            pltpu.SemaphoreType.DMA,
            pltpu.SemaphoreType.DMA,
            pltpu.SemaphoreType.DMA,
            pltpu.SemaphoreType.DMA,
        ],
    )(x, col3, row3, w3, zeros)


def _tc_body(p_ref, w_ref, o_ref):
    acc = p_ref[0] + p_ref[1]
    o_ref[...] = jnp.maximum(
        jnp.dot(acc, w_ref[...], preferred_element_type=jnp.float32), 0.0)


@jax.jit
def _tc_combine(p, W):
    bm = 1000
    return pl.pallas_call(
        _tc_body,
        grid=(N // bm,),
        in_specs=[
            pl.BlockSpec((NC, bm, D), lambda i: (0, i, 0)),
            pl.BlockSpec((D, D), lambda i: (0, 0)),
        ],
        out_specs=pl.BlockSpec((bm, D), lambda i: (i, 0)),
        out_shape=jax.ShapeDtypeStruct((N, D), jnp.float32),
    )(p, W)


def kernel(x, edge_index, edge_weight, W):
    row = edge_index[0]
    col = edge_index[1]
    pad = EPAD - E
    col_p = jnp.concatenate([col, jnp.zeros((pad,), jnp.int32)])
    row_p = jnp.concatenate([row, jnp.zeros((pad,), jnp.int32)])
    w_p = jnp.concatenate([edge_weight, jnp.zeros((pad,), jnp.float32)])
    col3 = col_p.reshape(NW, NCH, C)
    row3 = row_p.reshape(NW, NCH, C)
    w3 = w_p.reshape(NW, NCH, C)
    zeros = jnp.zeros((NPAD, D), jnp.float32)
    p = _sc_aggregate(x, col3, row3, w3, zeros)
    return _tc_combine(p[:, :N, :], W)


# speedup vs baseline: 9.4659x; 2.8448x over previous
import jax
import jax.numpy as jnp
from jax import lax
from jax.experimental import pallas as pl
from jax.experimental.pallas import tpu as pltpu
from jax.experimental.pallas import tpu_sc as plsc

N = 10000
E = 320000
D = 128
NC = 2
NS = 16
NW = NC * NS

C = 128
NCH = 80
EPT = C * NCH
EPAD = EPT * NW

NPAD = 10240
ROWS_PER_TILE = NPAD // NS

_GATHER_DNUMS = lax.GatherDimensionNumbers(
    offset_dims=(), collapsed_slice_dims=(0,), start_index_map=(0,))


def _scale_chunk(rows_b, w2, b):
    @pl.loop(0, C // 16)
    def _g(g):
        base = g * 16
        wvec = w2[b, pl.ds(base, 16)]
        for l in range(16):
            wb = lax.gather(
                wvec, jnp.full((16, 1), l, jnp.int32),
                _GATHER_DNUMS, slice_sizes=(1,),
                mode=lax.GatherScatterMode.PROMISE_IN_BOUNDS)
            for f in range(D // 16):
                sl = pl.ds(f * 16, 16)
                rows_b[base + l, sl] = rows_b[base + l, sl] * wb


def _edge_load(col_hbm, row_hbm, w_hbm, col2, row2, w2, wid, k, b, es):
    pltpu.async_copy(col_hbm.at[wid].at[k], col2.at[b], es)
    pltpu.async_copy(row_hbm.at[wid].at[k], row2.at[b], es)
    pltpu.async_copy(w_hbm.at[wid].at[k], w2.at[b], es)


def _edge_wait(col_hbm, col2, row2, w2, wid, b, es):
    pltpu.make_async_copy(col_hbm.at[0].at[0], col2.at[b], es).wait()
    pltpu.make_async_copy(col_hbm.at[0].at[0], row2.at[b], es).wait()
    pltpu.make_async_copy(col_hbm.at[0].at[0], w2.at[b], es).wait()


def _sc_body(x_hbm, col_hbm, row_hbm, w_hbm, zeros_hbm, out_hbm,
             col2, row2, w2, rows0, rows1, agg_sh, gs0, gs1, es0, es1):
    c = lax.axis_index("c")
    s = lax.axis_index("s")
    wid = c * NS + s

    pltpu.sync_copy(zeros_hbm.at[pl.ds(s * ROWS_PER_TILE, ROWS_PER_TILE)],
                    agg_sh.at[pl.ds(s * ROWS_PER_TILE, ROWS_PER_TILE)])

    plsc.subcore_barrier()

    xh = x_hbm
    ess = (es0, es1)

    _edge_load(col_hbm, row_hbm, w_hbm, col2, row2, w2, wid, 0, 0, es0)
    _edge_load(col_hbm, row_hbm, w_hbm, col2, row2, w2, wid, 1, 1, es1)
    _edge_wait(col_hbm, col2, row2, w2, wid, 0, es0)
    pltpu.async_copy(xh.at[col2.at[0]], rows0, gs0)

    @pl.loop(0, NCH, step=2)
    def _outer(j):
        for b in range(2):
            k = j + b
            o = 1 - b
            rows_b, rows_o = (rows0, rows1) if b == 0 else (rows1, rows0)
            gs_b, gs_o = (gs0, gs1) if b == 0 else (gs1, gs0)

            pltpu.make_async_copy(xh.at[col2.at[b]], rows_b, gs_b).wait()

            @pl.when(k + 1 < NCH)
            def _():
                _edge_wait(col_hbm, col2, row2, w2, wid, o, ess[o])
                pltpu.async_copy(xh.at[col2.at[o]], rows_o, gs_o)

            _scale_chunk(rows_b, w2, b)

            pltpu.sync_copy(rows_b, agg_sh.at[row2.at[b]], add=True)

            @pl.when(k + 2 < NCH)
            def _():
                _edge_load(col_hbm, row_hbm, w_hbm, col2, row2, w2,
                           wid, k + 2, b, ess[b])

    plsc.subcore_barrier()

    pltpu.sync_copy(agg_sh.at[pl.ds(s * ROWS_PER_TILE, ROWS_PER_TILE)],
                    out_hbm.at[c, pl.ds(s * ROWS_PER_TILE, ROWS_PER_TILE)])


@jax.jit
def _sc_aggregate(x, col3, row3, w3, zeros):
    mesh = plsc.VectorSubcoreMesh(core_axis_name="c", subcore_axis_name="s")
    return pl.kernel(
        _sc_body,
        out_type=jax.ShapeDtypeStruct((NC, NPAD, D), jnp.float32),
        mesh=mesh,
        scratch_types=[
            pltpu.VMEM((2, C), jnp.int32),
            pltpu.VMEM((2, C), jnp.int32),
            pltpu.VMEM((2, C), jnp.float32),
            pltpu.VMEM((C, D), jnp.float32),
            pltpu.VMEM((C, D), jnp.float32),
            pltpu.VMEM_SHARED((NPAD, D), jnp.float32),
            pltpu.SemaphoreType.DMA,
            pltpu.SemaphoreType.DMA,
            pltpu.SemaphoreType.DMA,
            pltpu.SemaphoreType.DMA,
        ],
    )(x, col3, row3, w3, zeros)


def _tc_body(p_ref, w_ref, o_ref):
    acc = p_ref[0] + p_ref[1]
    o_ref[...] = jnp.maximum(
        jnp.dot(acc, w_ref[...], preferred_element_type=jnp.float32), 0.0)


@jax.jit
def _tc_combine(p, W):
    bm = 1000
    return pl.pallas_call(
        _tc_body,
        grid=(N // bm,),
        in_specs=[
            pl.BlockSpec((NC, bm, D), lambda i: (0, i, 0)),
            pl.BlockSpec((D, D), lambda i: (0, 0)),
        ],
        out_specs=pl.BlockSpec((bm, D), lambda i: (i, 0)),
        out_shape=jax.ShapeDtypeStruct((N, D), jnp.float32),
    )(p, W)


def kernel(x, edge_index, edge_weight, W):
    row = edge_index[0]
    col = edge_index[1]
    pad = EPAD - E
    spread = jnp.arange(pad, dtype=jnp.int32)
    col_p = jnp.concatenate([col, spread % N])
    row_p = jnp.concatenate([row, spread % NPAD])
    w_p = jnp.concatenate([edge_weight, jnp.zeros((pad,), jnp.float32)])
    col3 = col_p.reshape(NW, NCH, C)
    row3 = row_p.reshape(NW, NCH, C)
    w3 = w_p.reshape(NW, NCH, C)
    zeros = jnp.zeros((NPAD, D), jnp.float32)
    p = _sc_aggregate(x, col3, row3, w3, zeros)
    return _tc_combine(p[:, :N, :], W)


# speedup vs baseline: 10.7355x; 1.1341x over previous
import jax
import jax.numpy as jnp
from jax import lax
from jax.experimental import pallas as pl
from jax.experimental.pallas import tpu as pltpu
from jax.experimental.pallas import tpu_sc as plsc

N = 10000
E = 320000
D = 128
NC = 2
NS = 16
NW = NC * NS

C = 128
NCH = 80
EPT = C * NCH
EPAD = EPT * NW

NPAD = 10240
ROWS_PER_TILE = NPAD // NS

_GATHER_DNUMS = lax.GatherDimensionNumbers(
    offset_dims=(), collapsed_slice_dims=(0,), start_index_map=(0,))


def _scale_chunk(rows_b, w2, b):
    @pl.loop(0, C // 16)
    def _g(g):
        base = g * 16
        wvec = w2[b, pl.ds(base, 16)]
        for l in range(16):
            wb = lax.gather(
                wvec, jnp.full((16, 1), l, jnp.int32),
                _GATHER_DNUMS, slice_sizes=(1,),
                mode=lax.GatherScatterMode.PROMISE_IN_BOUNDS)
            for f in range(D // 16):
                sl = pl.ds(f * 16, 16)
                rows_b[base + l, sl] = rows_b[base + l, sl] * wb


def _edge_load(col_hbm, row_hbm, w_hbm, col4, row4, w4, wid, k, slot, es):
    pltpu.async_copy(col_hbm.at[wid].at[k], col4.at[slot], es)
    pltpu.async_copy(row_hbm.at[wid].at[k], row4.at[slot], es)
    pltpu.async_copy(w_hbm.at[wid].at[k], w4.at[slot], es)


def _edge_wait(col_hbm, col4, row4, w4, slot, es):
    pltpu.make_async_copy(col_hbm.at[0].at[0], col4.at[slot], es).wait()
    pltpu.make_async_copy(col_hbm.at[0].at[0], row4.at[slot], es).wait()
    pltpu.make_async_copy(col_hbm.at[0].at[0], w4.at[slot], es).wait()


def _sc_body(x_hbm, col_hbm, row_hbm, w_hbm, zeros_hbm, out_hbm,
             col4, row4, w4, rows0, rows1, agg_sh,
             gs0, gs1, ss0, ss1, es0, es1, es2, es3):
    c = lax.axis_index("c")
    s = lax.axis_index("s")
    wid = c * NS + s

    pltpu.sync_copy(zeros_hbm.at[pl.ds(s * ROWS_PER_TILE, ROWS_PER_TILE)],
                    agg_sh.at[pl.ds(s * ROWS_PER_TILE, ROWS_PER_TILE)])

    plsc.subcore_barrier()

    xh = x_hbm
    rows = (rows0, rows1)
    gss = (gs0, gs1)
    sss = (ss0, ss1)
    ess = (es0, es1, es2, es3)

    for m in range(3):
        _edge_load(col_hbm, row_hbm, w_hbm, col4, row4, w4, wid, m, m,
                   ess[m])
    _edge_wait(col_hbm, col4, row4, w4, 0, ess[0])
    pltpu.async_copy(xh.at[col4.at[0]], rows0, gs0)

    @pl.loop(0, NCH, step=4)
    def _outer(j):
        for b in range(4):
            k = j + b
            B2 = b % 2
            B4 = b
            S3 = (b + 3) % 4

            pltpu.make_async_copy(xh.at[col4.at[B4]], rows[B2], gss[B2]).wait()

            @pl.when(k + 1 < NCH)
            def _():
                @pl.when(k >= 1)
                def _():
                    pltpu.make_async_copy(
                        rows[1 - B2], agg_sh.at[row4.at[S3]],
                        sss[1 - B2]).wait()
                _edge_wait(col_hbm, col4, row4, w4, (b + 1) % 4,
                           ess[(b + 1) % 4])
                pltpu.async_copy(xh.at[col4.at[(b + 1) % 4]], rows[1 - B2],
                                 gss[1 - B2])

            @pl.when(k + 3 < NCH)
            def _():
                _edge_load(col_hbm, row_hbm, w_hbm, col4, row4, w4,
                           wid, k + 3, S3, ess[S3])

            _scale_chunk(rows[B2], w4, B4)

            pltpu.async_copy(rows[B2], agg_sh.at[row4.at[B4]], sss[B2],
                             add=True)

    pltpu.make_async_copy(rows0, agg_sh.at[row4.at[0]], ss0).wait()
    pltpu.make_async_copy(rows1, agg_sh.at[row4.at[1]], ss1).wait()

    plsc.subcore_barrier()

    pltpu.sync_copy(agg_sh.at[pl.ds(s * ROWS_PER_TILE, ROWS_PER_TILE)],
                    out_hbm.at[c, pl.ds(s * ROWS_PER_TILE, ROWS_PER_TILE)])


@jax.jit
def _sc_aggregate(x, col3, row3, w3, zeros):
    mesh = plsc.VectorSubcoreMesh(core_axis_name="c", subcore_axis_name="s")
    return pl.kernel(
        _sc_body,
        out_type=jax.ShapeDtypeStruct((NC, NPAD, D), jnp.float32),
        mesh=mesh,
        scratch_types=[
            pltpu.VMEM((4, C), jnp.int32),
            pltpu.VMEM((4, C), jnp.int32),
            pltpu.VMEM((4, C), jnp.float32),
            pltpu.VMEM((C, D), jnp.float32),
            pltpu.VMEM((C, D), jnp.float32),
            pltpu.VMEM_SHARED((NPAD, D), jnp.float32),
            pltpu.SemaphoreType.DMA,
            pltpu.SemaphoreType.DMA,
            pltpu.SemaphoreType.DMA,
            pltpu.SemaphoreType.DMA,
            pltpu.SemaphoreType.DMA,
            pltpu.SemaphoreType.DMA,
            pltpu.SemaphoreType.DMA,
            pltpu.SemaphoreType.DMA,
        ],
    )(x, col3, row3, w3, zeros)


def _tc_body(p_ref, w_ref, o_ref):
    acc = p_ref[0] + p_ref[1]
    o_ref[...] = jnp.maximum(
        jnp.dot(acc, w_ref[...], preferred_element_type=jnp.float32), 0.0)


@jax.jit
def _tc_combine(p, W):
    bm = 1000
    return pl.pallas_call(
        _tc_body,
        grid=(N // bm,),
        in_specs=[
            pl.BlockSpec((NC, bm, D), lambda i: (0, i, 0)),
            pl.BlockSpec((D, D), lambda i: (0, 0)),
        ],
        out_specs=pl.BlockSpec((bm, D), lambda i: (i, 0)),
        out_shape=jax.ShapeDtypeStruct((N, D), jnp.float32),
    )(p, W)


def kernel(x, edge_index, edge_weight, W):
    row = edge_index[0]
    col = edge_index[1]
    pad = EPAD - E
    spread = jnp.arange(pad, dtype=jnp.int32)
    col_p = jnp.concatenate([col, spread % N])
    row_p = jnp.concatenate([row, spread % NPAD])
    w_p = jnp.concatenate([edge_weight, jnp.zeros((pad,), jnp.float32)])
    col3 = col_p.reshape(NW, NCH, C)
    row3 = row_p.reshape(NW, NCH, C)
    w3 = w_p.reshape(NW, NCH, C)
    zeros = jnp.zeros((NPAD, D), jnp.float32)
    p = _sc_aggregate(x, col3, row3, w3, zeros)
    return _tc_combine(p[:, :N, :], W)


# speedup vs baseline: 12.0850x; 1.1257x over previous
import jax
import jax.numpy as jnp
from jax import lax
from jax.experimental import pallas as pl
from jax.experimental.pallas import tpu as pltpu
from jax.experimental.pallas import tpu_sc as plsc

N = 10000
E = 320000
D = 128
NC = 2
NS = 16
NW = NC * NS

C = 80
NCH = 128
EPT = C * NCH
EPAD = EPT * NW

NPAD = 10240
ROWS_PER_TILE = NPAD // NS

_GATHER_DNUMS = lax.GatherDimensionNumbers(
    offset_dims=(), collapsed_slice_dims=(0,), start_index_map=(0,))


def _scale_chunk(rows_b, w8, slot):
    @pl.loop(0, C // 16)
    def _g(g):
        base = g * 16
        wvec = w8[slot, pl.ds(base, 16)]
        for l in range(16):
            wb = lax.gather(
                wvec, jnp.full((16, 1), l, jnp.int32),
                _GATHER_DNUMS, slice_sizes=(1,),
                mode=lax.GatherScatterMode.PROMISE_IN_BOUNDS)
            for f in range(D // 16):
                sl = pl.ds(f * 16, 16)
                rows_b[base + l, sl] = rows_b[base + l, sl] * wb


def _edge_load(col_hbm, row_hbm, w_hbm, col8, row8, w8, wid, k, slot, es):
    pltpu.async_copy(col_hbm.at[wid].at[k], col8.at[slot], es)
    pltpu.async_copy(row_hbm.at[wid].at[k], row8.at[slot], es)
    pltpu.async_copy(w_hbm.at[wid].at[k], w8.at[slot], es)


def _edge_wait(col_hbm, col8, row8, w8, slot, es):
    pltpu.make_async_copy(col_hbm.at[0].at[0], col8.at[slot], es).wait()
    pltpu.make_async_copy(col_hbm.at[0].at[0], row8.at[slot], es).wait()
    pltpu.make_async_copy(col_hbm.at[0].at[0], w8.at[slot], es).wait()


def _sc_body(x_hbm, col_hbm, row_hbm, w_hbm, zeros_hbm, out_hbm,
             col8, row8, w8, rows0, rows1, rows2, rows3, agg_sh,
             gs0, gs1, gs2, gs3, ss0, ss1, ss2, ss3,
             es0, es1, es2, es3, es4, es5, es6, es7):
    c = lax.axis_index("c")
    s = lax.axis_index("s")
    wid = c * NS + s

    pltpu.sync_copy(zeros_hbm.at[pl.ds(s * ROWS_PER_TILE, ROWS_PER_TILE)],
                    agg_sh.at[pl.ds(s * ROWS_PER_TILE, ROWS_PER_TILE)])

    plsc.subcore_barrier()

    xh = x_hbm
    rows = (rows0, rows1, rows2, rows3)
    gss = (gs0, gs1, gs2, gs3)
    sss = (ss0, ss1, ss2, ss3)
    ess = (es0, es1, es2, es3, es4, es5, es6, es7)

    for m in range(4):
        _edge_load(col_hbm, row_hbm, w_hbm, col8, row8, w8, wid, m, m,
                   ess[m])
    for m in range(2):
        _edge_wait(col_hbm, col8, row8, w8, m, ess[m])
        pltpu.async_copy(xh.at[col8.at[m]], rows[m], gss[m])

    @pl.loop(0, NCH, step=8)
    def _outer(j):
        for b in range(8):
            k = j + b
            B4 = b % 4
            B8 = b
            A4 = (b + 2) % 4
            A8 = (b + 2) % 8
            L8 = (b + 4) % 8

            @pl.when(k >= 2)
            def _():
                pltpu.make_async_copy(
                    rows[A4], agg_sh.at[row8.at[A8]], sss[A4]).wait()

            @pl.when(k + 2 < NCH)
            def _():
                _edge_wait(col_hbm, col8, row8, w8, A8, ess[A8])
                pltpu.async_copy(xh.at[col8.at[A8]], rows[A4], gss[A4])

            @pl.when(k + 4 < NCH)
            def _():
                _edge_load(col_hbm, row_hbm, w_hbm, col8, row8, w8,
                           wid, k + 4, L8, ess[L8])

            pltpu.make_async_copy(xh.at[col8.at[B8]], rows[B4], gss[B4]).wait()

            _scale_chunk(rows[B4], w8, B8)

            pltpu.async_copy(rows[B4], agg_sh.at[row8.at[B8]], sss[B4],
                             add=True)

    pltpu.make_async_copy(rows[(NCH - 2) % 4],
                          agg_sh.at[row8.at[(NCH - 2) % 8]],
                          sss[(NCH - 2) % 4]).wait()
    pltpu.make_async_copy(rows[(NCH - 1) % 4],
                          agg_sh.at[row8.at[(NCH - 1) % 8]],
                          sss[(NCH - 1) % 4]).wait()

    plsc.subcore_barrier()

    pltpu.sync_copy(agg_sh.at[pl.ds(s * ROWS_PER_TILE, ROWS_PER_TILE)],
                    out_hbm.at[c, pl.ds(s * ROWS_PER_TILE, ROWS_PER_TILE)])


@jax.jit
def _sc_aggregate(x, col3, row3, w3, zeros):
    mesh = plsc.VectorSubcoreMesh(core_axis_name="c", subcore_axis_name="s")
    return pl.kernel(
        _sc_body,
        out_type=jax.ShapeDtypeStruct((NC, NPAD, D), jnp.float32),
        mesh=mesh,
        scratch_types=[
            pltpu.VMEM((8, C), jnp.int32),
            pltpu.VMEM((8, C), jnp.int32),
            pltpu.VMEM((8, C), jnp.float32),
            pltpu.VMEM((C, D), jnp.float32),
            pltpu.VMEM((C, D), jnp.float32),
            pltpu.VMEM((C, D), jnp.float32),
            pltpu.VMEM((C, D), jnp.float32),
            pltpu.VMEM_SHARED((NPAD, D), jnp.float32),
            pltpu.SemaphoreType.DMA,
            pltpu.SemaphoreType.DMA,
            pltpu.SemaphoreType.DMA,
            pltpu.SemaphoreType.DMA,
            pltpu.SemaphoreType.DMA,
            pltpu.SemaphoreType.DMA,
            pltpu.SemaphoreType.DMA,
            pltpu.SemaphoreType.DMA,
            pltpu.SemaphoreType.DMA,
            pltpu.SemaphoreType.DMA,
            pltpu.SemaphoreType.DMA,
            pltpu.SemaphoreType.DMA,
            pltpu.SemaphoreType.DMA,
            pltpu.SemaphoreType.DMA,
            pltpu.SemaphoreType.DMA,
            pltpu.SemaphoreType.DMA,
        ],
    )(x, col3, row3, w3, zeros)


def _tc_body(p_ref, w_ref, o_ref):
    acc = p_ref[0] + p_ref[1]
    o_ref[...] = jnp.maximum(
        jnp.dot(acc, w_ref[...], preferred_element_type=jnp.float32), 0.0)


@jax.jit
def _tc_combine(p, W):
    bm = 1000
    return pl.pallas_call(
        _tc_body,
        grid=(N // bm,),
        in_specs=[
            pl.BlockSpec((NC, bm, D), lambda i: (0, i, 0)),
            pl.BlockSpec((D, D), lambda i: (0, 0)),
        ],
        out_specs=pl.BlockSpec((bm, D), lambda i: (i, 0)),
        out_shape=jax.ShapeDtypeStruct((N, D), jnp.float32),
    )(p, W)


def kernel(x, edge_index, edge_weight, W):
    row = edge_index[0]
    col = edge_index[1]
    pad = EPAD - E
    spread = jnp.arange(pad, dtype=jnp.int32)
    col_p = jnp.concatenate([col, spread % N])
    row_p = jnp.concatenate([row, spread % NPAD])
    w_p = jnp.concatenate([edge_weight, jnp.zeros((pad,), jnp.float32)])
    col3 = col_p.reshape(NW, NCH, C)
    row3 = row_p.reshape(NW, NCH, C)
    w3 = w_p.reshape(NW, NCH, C)
    zeros = jnp.zeros((NPAD, D), jnp.float32)
    p = _sc_aggregate(x, col3, row3, w3, zeros)
    return _tc_combine(p, W)


# speedup vs baseline: 12.3887x; 1.0251x over previous
import jax
import jax.numpy as jnp
from jax import lax
from jax.experimental import pallas as pl
from jax.experimental.pallas import tpu as pltpu
from jax.experimental.pallas import tpu_sc as plsc

N = 10000
E = 320000
D = 128
NC = 2
NS = 16
NW = NC * NS

C = 80
NCH = 128
EPT = C * NCH
EPAD = EPT * NW

NTAIL = 8
NMAIN = NCH - NTAIL

NPAD = 10240
ROWS_PER_TILE = NPAD // NS

_GATHER_DNUMS = lax.GatherDimensionNumbers(
    offset_dims=(), collapsed_slice_dims=(0,), start_index_map=(0,))


def _scale_chunk(rows_b, w8, slot):
    @pl.loop(0, C // 16)
    def _g(g):
        base = g * 16
        wvec = w8[slot, pl.ds(base, 16)]
        for l in range(16):
            wb = lax.gather(
                wvec, jnp.full((16, 1), l, jnp.int32),
                _GATHER_DNUMS, slice_sizes=(1,),
                mode=lax.GatherScatterMode.PROMISE_IN_BOUNDS)
            for f in range(D // 16):
                sl = pl.ds(f * 16, 16)
                rows_b[base + l, sl] = rows_b[base + l, sl] * wb


def _edge_load(col_hbm, row_hbm, w_hbm, col8, row8, w8, wid, k, slot, es):
    pltpu.async_copy(col_hbm.at[wid].at[k], col8.at[slot], es)
    pltpu.async_copy(row_hbm.at[wid].at[k], row8.at[slot], es)
    pltpu.async_copy(w_hbm.at[wid].at[k], w8.at[slot], es)


def _edge_wait(col_hbm, col8, row8, w8, slot, es):
    pltpu.make_async_copy(col_hbm.at[0].at[0], col8.at[slot], es).wait()
    pltpu.make_async_copy(col_hbm.at[0].at[0], row8.at[slot], es).wait()
    pltpu.make_async_copy(col_hbm.at[0].at[0], w8.at[slot], es).wait()


def _sc_body(x_hbm, col_hbm, row_hbm, w_hbm, out_hbm,
             col8, row8, w8, rows0, rows1, rows2, rows3, agg_sh,
             gs0, gs1, gs2, gs3, ss0, ss1, ss2, ss3,
             es0, es1, es2, es3, es4, es5, es6, es7):
    c = lax.axis_index("c")
    s = lax.axis_index("s")
    wid = c * NS + s

    rows = (rows0, rows1, rows2, rows3)
    gss = (gs0, gs1, gs2, gs3)
    sss = (ss0, ss1, ss2, ss3)
    ess = (es0, es1, es2, es3, es4, es5, es6, es7)

    @pl.loop(0, C)
    def _z(i):
        for f in range(D // 16):
            rows0[i, pl.ds(f * 16, 16)] = jnp.zeros((16,), jnp.float32)
    for t in range(ROWS_PER_TILE // C):
        pltpu.async_copy(
            rows0, agg_sh.at[pl.ds(s * ROWS_PER_TILE + t * C, C)], ess[t])
    for t in range(ROWS_PER_TILE // C):
        pltpu.make_async_copy(
            rows0, agg_sh.at[pl.ds(s * ROWS_PER_TILE + t * C, C)],
            ess[t]).wait()

    plsc.subcore_barrier()

    xh = x_hbm

    def chunk_step(k, b, static_k=None):
        kk = k if static_k is None else static_k
        B4 = b % 4
        B8 = b
        A4 = (b + 2) % 4
        A8 = (b + 2) % 8
        L8 = (b + 4) % 8

        def wait_scatter():
            pltpu.make_async_copy(
                rows[A4], agg_sh.at[row8.at[A8]], sss[A4]).wait()

        def launch_gather():
            _edge_wait(col_hbm, col8, row8, w8, A8, ess[A8])
            pltpu.async_copy(xh.at[col8.at[A8]], rows[A4], gss[A4])

        def load_edges():
            _edge_load(col_hbm, row_hbm, w_hbm, col8, row8, w8,
                       wid, kk + 4, L8, ess[L8])

        if static_k is None:
            @pl.when(kk >= 2)
            def _():
                wait_scatter()
            launch_gather()
            load_edges()
        else:
            if static_k >= 2:
                wait_scatter()
            if static_k + 2 < NCH:
                launch_gather()
            if static_k + 4 < NCH:
                load_edges()

        pltpu.make_async_copy(xh.at[col8.at[B8]], rows[B4], gss[B4]).wait()

        _scale_chunk(rows[B4], w8, B8)

        pltpu.async_copy(rows[B4], agg_sh.at[row8.at[B8]], sss[B4],
                         add=True)

    for m in range(4):
        _edge_load(col_hbm, row_hbm, w_hbm, col8, row8, w8, wid, m, m,
                   ess[m])
    for m in range(2):
        _edge_wait(col_hbm, col8, row8, w8, m, ess[m])
        pltpu.async_copy(xh.at[col8.at[m]], rows[m], gss[m])

    @pl.loop(0, NMAIN, step=8)
    def _outer(j):
        for b in range(8):
            chunk_step(j + b, b)

    for k in range(NMAIN, NCH):
        chunk_step(k, k % 8, static_k=k)

    pltpu.make_async_copy(rows[(NCH - 2) % 4],
                          agg_sh.at[row8.at[(NCH - 2) % 8]],
                          sss[(NCH - 2) % 4]).wait()
    pltpu.make_async_copy(rows[(NCH - 1) % 4],
                          agg_sh.at[row8.at[(NCH - 1) % 8]],
                          sss[(NCH - 1) % 4]).wait()

    plsc.subcore_barrier()

    pltpu.sync_copy(agg_sh.at[pl.ds(s * ROWS_PER_TILE, ROWS_PER_TILE)],
                    out_hbm.at[c, pl.ds(s * ROWS_PER_TILE, ROWS_PER_TILE)])


@jax.jit
def _sc_aggregate(x, col3, row3, w3):
    mesh = plsc.VectorSubcoreMesh(core_axis_name="c", subcore_axis_name="s")
    return pl.kernel(
        _sc_body,
        out_type=jax.ShapeDtypeStruct((NC, NPAD, D), jnp.float32),
        mesh=mesh,
        scratch_types=[
            pltpu.VMEM((8, C), jnp.int32),
            pltpu.VMEM((8, C), jnp.int32),
            pltpu.VMEM((8, C), jnp.float32),
            pltpu.VMEM((C, D), jnp.float32),
            pltpu.VMEM((C, D), jnp.float32),
            pltpu.VMEM((C, D), jnp.float32),
            pltpu.VMEM((C, D), jnp.float32),
            pltpu.VMEM_SHARED((NPAD, D), jnp.float32),
            pltpu.SemaphoreType.DMA,
            pltpu.SemaphoreType.DMA,
            pltpu.SemaphoreType.DMA,
            pltpu.SemaphoreType.DMA,
            pltpu.SemaphoreType.DMA,
            pltpu.SemaphoreType.DMA,
            pltpu.SemaphoreType.DMA,
            pltpu.SemaphoreType.DMA,
            pltpu.SemaphoreType.DMA,
            pltpu.SemaphoreType.DMA,
            pltpu.SemaphoreType.DMA,
            pltpu.SemaphoreType.DMA,
            pltpu.SemaphoreType.DMA,
            pltpu.SemaphoreType.DMA,
            pltpu.SemaphoreType.DMA,
            pltpu.SemaphoreType.DMA,
        ],
    )(x, col3, row3, w3)


def _tc_body(p_ref, w_ref, o_ref):
    acc = p_ref[0] + p_ref[1]
    o_ref[...] = jnp.maximum(
        jnp.dot(acc, w_ref[...], preferred_element_type=jnp.float32), 0.0)


@jax.jit
def _tc_combine(p, W):
    bm = 1000
    return pl.pallas_call(
        _tc_body,
        grid=(N // bm,),
        in_specs=[
            pl.BlockSpec((NC, bm, D), lambda i: (0, i, 0)),
            pl.BlockSpec((D, D), lambda i: (0, 0)),
        ],
        out_specs=pl.BlockSpec((bm, D), lambda i: (i, 0)),
        out_shape=jax.ShapeDtypeStruct((N, D), jnp.float32),
    )(p, W)


def kernel(x, edge_index, edge_weight, W):
    pad = EPAD - E
    spread = jnp.arange(pad, dtype=jnp.int32)
    col_p = jnp.concatenate([edge_index[1], spread % N])
    row_p = jnp.concatenate([edge_index[0], spread % NPAD])
    w_p = jnp.concatenate([edge_weight, jnp.zeros((pad,), jnp.float32)])
    col3 = col_p.reshape(NW, NCH, C)
    row3 = row_p.reshape(NW, NCH, C)
    w3 = w_p.reshape(NW, NCH, C)
    p = _sc_aggregate(x, col3, row3, w3)
    return _tc_combine(p, W)


# speedup vs baseline: 12.3911x; 1.0002x over previous
import jax
import jax.numpy as jnp
import numpy as np
from jax import lax
from jax.experimental import pallas as pl
from jax.experimental.pallas import tpu as pltpu
from jax.experimental.pallas import tpu_sc as plsc

N = 10000
E = 320000
D = 128
NC = 2
NS = 16
NW = NC * NS

C = 80
NCH = 128
EPT = C * NCH
EPAD = EPT * NW

NTAIL = 8
NMAIN = NCH - NTAIL

NPAD = 10240
ROWS_PER_TILE = NPAD // NS

_GATHER_DNUMS = lax.GatherDimensionNumbers(
    offset_dims=(), collapsed_slice_dims=(0,), start_index_map=(0,))


def _scale_chunk(rows_b, w8, slot):
    @pl.loop(0, C // 16)
    def _g(g):
        base = g * 16
        wvec = w8[slot, pl.ds(base, 16)]
        for l in range(16):
            wb = lax.gather(
                wvec, jnp.full((16, 1), l, jnp.int32),
                _GATHER_DNUMS, slice_sizes=(1,),
                mode=lax.GatherScatterMode.PROMISE_IN_BOUNDS)
            for f in range(D // 16):
                sl = pl.ds(f * 16, 16)
                rows_b[base + l, sl] = rows_b[base + l, sl] * wb


def _edge_load(col_hbm, row_hbm, w_hbm, col8, row8, w8, wid, k, slot, es):
    pltpu.async_copy(col_hbm.at[wid].at[k], col8.at[slot], es)
    pltpu.async_copy(row_hbm.at[wid].at[k], row8.at[slot], es)
    pltpu.async_copy(w_hbm.at[wid].at[k], w8.at[slot], es)


def _edge_wait(col_hbm, col8, row8, w8, slot, es):
    pltpu.make_async_copy(col_hbm.at[0].at[0], col8.at[slot], es).wait()
    pltpu.make_async_copy(col_hbm.at[0].at[0], row8.at[slot], es).wait()
    pltpu.make_async_copy(col_hbm.at[0].at[0], w8.at[slot], es).wait()


def _sc_body(x_hbm, col_hbm, row_hbm, w_hbm, out_hbm,
             col8, row8, w8, rows0, rows1, rows2, rows3, agg_sh,
             gs0, gs1, gs2, gs3, ss0, ss1, ss2, ss3,
             es0, es1, es2, es3, es4, es5, es6, es7):
    c = lax.axis_index("c")
    s = lax.axis_index("s")
    wid = c * NS + s

    rows = (rows0, rows1, rows2, rows3)
    gss = (gs0, gs1, gs2, gs3)
    sss = (ss0, ss1, ss2, ss3)
    ess = (es0, es1, es2, es3, es4, es5, es6, es7)

    @pl.loop(0, C)
    def _z(i):
        for f in range(D // 16):
            rows0[i, pl.ds(f * 16, 16)] = jnp.zeros((16,), jnp.float32)
    for t in range(ROWS_PER_TILE // C):
        pltpu.async_copy(
            rows0, agg_sh.at[pl.ds(s * ROWS_PER_TILE + t * C, C)], ess[t])
    for t in range(ROWS_PER_TILE // C):
        pltpu.make_async_copy(
            rows0, agg_sh.at[pl.ds(s * ROWS_PER_TILE + t * C, C)],
            ess[t]).wait()

    plsc.subcore_barrier()

    xh = x_hbm

    def chunk_step(k, b, static_k=None):
        kk = k if static_k is None else static_k
        B4 = b % 4
        B8 = b
        A4 = (b + 2) % 4
        A8 = (b + 2) % 8
        L8 = (b + 4) % 8

        def wait_scatter():
            pltpu.make_async_copy(
                rows[A4], agg_sh.at[row8.at[A8]], sss[A4]).wait()

        def launch_gather():
            _edge_wait(col_hbm, col8, row8, w8, A8, ess[A8])
            pltpu.async_copy(xh.at[col8.at[A8]], rows[A4], gss[A4])

        def load_edges():
            _edge_load(col_hbm, row_hbm, w_hbm, col8, row8, w8,
                       wid, kk + 4, L8, ess[L8])

        if static_k is None:
            @pl.when(kk >= 2)
            def _():
                wait_scatter()
            launch_gather()
            load_edges()
        else:
            if static_k >= 2:
                wait_scatter()
            if static_k + 2 < NCH:
                launch_gather()
            if static_k + 4 < NCH:
                load_edges()

        pltpu.make_async_copy(xh.at[col8.at[B8]], rows[B4], gss[B4]).wait()

        _scale_chunk(rows[B4], w8, B8)

        pltpu.async_copy(rows[B4], agg_sh.at[row8.at[B8]], sss[B4],
                         add=True)

    for m in range(4):
        _edge_load(col_hbm, row_hbm, w_hbm, col8, row8, w8, wid, m, m,
                   ess[m])
    for m in range(2):
        _edge_wait(col_hbm, col8, row8, w8, m, ess[m])
        pltpu.async_copy(xh.at[col8.at[m]], rows[m], gss[m])

    @pl.loop(0, NMAIN, step=8)
    def _outer(j):
        for b in range(8):
            chunk_step(j + b, b)

    for k in range(NMAIN, NCH):
        chunk_step(k, k % 8, static_k=k)

    pltpu.make_async_copy(rows[(NCH - 2) % 4],
                          agg_sh.at[row8.at[(NCH - 2) % 8]],
                          sss[(NCH - 2) % 4]).wait()
    pltpu.make_async_copy(rows[(NCH - 1) % 4],
                          agg_sh.at[row8.at[(NCH - 1) % 8]],
                          sss[(NCH - 1) % 4]).wait()

    plsc.subcore_barrier()

    pltpu.sync_copy(agg_sh.at[pl.ds(s * ROWS_PER_TILE, ROWS_PER_TILE)],
                    out_hbm.at[c, pl.ds(s * ROWS_PER_TILE, ROWS_PER_TILE)])


@jax.jit
def _sc_aggregate(x, col3, row3, w3):
    mesh = plsc.VectorSubcoreMesh(core_axis_name="c", subcore_axis_name="s")
    return pl.kernel(
        _sc_body,
        out_type=jax.ShapeDtypeStruct((NC, NPAD, D), jnp.float32),
        mesh=mesh,
        scratch_types=[
            pltpu.VMEM((8, C), jnp.int32),
            pltpu.VMEM((8, C), jnp.int32),
            pltpu.VMEM((8, C), jnp.float32),
            pltpu.VMEM((C, D), jnp.float32),
            pltpu.VMEM((C, D), jnp.float32),
            pltpu.VMEM((C, D), jnp.float32),
            pltpu.VMEM((C, D), jnp.float32),
            pltpu.VMEM_SHARED((NPAD, D), jnp.float32),
            pltpu.SemaphoreType.DMA,
            pltpu.SemaphoreType.DMA,
            pltpu.SemaphoreType.DMA,
            pltpu.SemaphoreType.DMA,
            pltpu.SemaphoreType.DMA,
            pltpu.SemaphoreType.DMA,
            pltpu.SemaphoreType.DMA,
            pltpu.SemaphoreType.DMA,
            pltpu.SemaphoreType.DMA,
            pltpu.SemaphoreType.DMA,
            pltpu.SemaphoreType.DMA,
            pltpu.SemaphoreType.DMA,
            pltpu.SemaphoreType.DMA,
            pltpu.SemaphoreType.DMA,
            pltpu.SemaphoreType.DMA,
            pltpu.SemaphoreType.DMA,
        ],
    )(x, col3, row3, w3)


def _tc_body(p_ref, w_ref, o_ref):
    acc = p_ref[0] + p_ref[1]
    o_ref[...] = jnp.maximum(
        jnp.dot(acc, w_ref[...], preferred_element_type=jnp.float32), 0.0)


@jax.jit
def _tc_combine(p, W):
    bm = 1000
    return pl.pallas_call(
        _tc_body,
        grid=(N // bm,),
        in_specs=[
            pl.BlockSpec((NC, bm, D), lambda i: (0, i, 0)),
            pl.BlockSpec((D, D), lambda i: (0, 0)),
        ],
        out_specs=pl.BlockSpec((bm, D), lambda i: (i, 0)),
        out_shape=jax.ShapeDtypeStruct((N, D), jnp.float32),
    )(p, W)


_PAD = EPAD - E
_PAD_COL = np.arange(_PAD, dtype=np.int32) % N
_PAD_ROW = np.arange(_PAD, dtype=np.int32) % NPAD
_PAD_W = np.zeros(_PAD, np.float32)


def kernel(x, edge_index, edge_weight, W):
    col_p = jnp.concatenate([edge_index[1], jnp.asarray(_PAD_COL)])
    row_p = jnp.concatenate([edge_index[0], jnp.asarray(_PAD_ROW)])
    w_p = jnp.concatenate([edge_weight, jnp.asarray(_PAD_W)])
    col3 = col_p.reshape(NW, NCH, C)
    row3 = row_p.reshape(NW, NCH, C)
    w3 = w_p.reshape(NW, NCH, C)
    p = _sc_aggregate(x, col3, row3, w3)
    return _tc_combine(p, W)
